# Initial kernel scaffold; baseline (speedup 1.0000x reference)
import jax
import jax.numpy as jnp
from jax.experimental import pallas as pl


def kernel(pe_k_weight, seq_len):
    raise NotImplementedError("write your pallas kernel here")


# speedup vs baseline: 8.1044x; 8.1044x over previous
import functools

import jax
import jax.numpy as jnp
from jax import lax
from jax.experimental import pallas as pl
from jax.experimental.pallas import tpu as pltpu
from jax.experimental.pallas import tpu_sc as plsc

_MAXLEN = 2048
_HEAD_DIM = 32
_TW = 2 * _MAXLEN * _HEAD_DIM
_ROW_W = _MAXLEN * _HEAD_DIM

_info = plsc.get_sparse_core_info()
_NC = _info.num_cores
_NS = _info.num_subcores
_NW = _NC * _NS
_LANES = 16

_ROWS_PER_W = _MAXLEN // _NW
_SPAN = _ROW_W + (_ROWS_PER_W - 1) * _HEAD_DIM


@functools.partial(
    pl.kernel,
    mesh=plsc.VectorSubcoreMesh(core_axis_name="c", subcore_axis_name="s"),
    out_type=jax.ShapeDtypeStruct((_MAXLEN * _ROW_W,), jnp.float32),
    scratch_types=[
        pltpu.VMEM((_SPAN,), jnp.float32),
    ],
)
def _rel_pos_sc(table_hbm, out_hbm, vbuf):
    cid = lax.axis_index("c")
    sid = lax.axis_index("s")
    wid = sid * _NC + cid
    row0 = wid * _ROWS_PER_W

    span_base = (_MAXLEN - row0 - (_ROWS_PER_W - 1)) * _HEAD_DIM
    pltpu.sync_copy(table_hbm.at[pl.ds(span_base, _SPAN)], vbuf)

    def clip_body(k, _):
        off = pl.multiple_of(k * _LANES, _LANES)
        v = vbuf[pl.ds(off, _LANES)]
        vbuf[pl.ds(off, _LANES)] = jnp.minimum(jnp.maximum(v, -5.0), 5.0)
        return 0

    lax.fori_loop(0, _SPAN // _LANES, clip_body, 0)

    def row_body(r, _):
        i = row0 + r
        local = (_ROWS_PER_W - 1 - r) * _HEAD_DIM
        pltpu.sync_copy(
            vbuf.at[pl.ds(local, _ROW_W)],
            out_hbm.at[pl.ds(i * _ROW_W, _ROW_W)],
        )
        return 0

    lax.fori_loop(0, _ROWS_PER_W, row_body, 0)


def kernel(pe_k_weight, seq_len):
    del seq_len
    flat = pe_k_weight.reshape(_TW)
    out = _rel_pos_sc(flat)
    return out.reshape(_MAXLEN, _MAXLEN, _HEAD_DIM)
